# baseline (speedup 1.0000x reference)
import functools

import jax
import jax.numpy as jnp
from jax.experimental import pallas as pl
from jax.experimental.pallas import tpu as pltpu


def _mmh_block(are_ref, moves_ref, mask_ref, g_ref,
               wq1_ref, bq1_ref, wq2_ref, bq2_ref, wkey_ref, bkey_ref,
               logits_ref, policy_ref, idx_ref, *, n_moves, tile):
    f32 = jnp.float32
    bt = logits_ref.shape[0]
    h = jnp.dot(are_ref[...], wq1_ref[...], preferred_element_type=f32)
    h = jnp.maximum(h + bq1_ref[...], 0.0)
    q = jnp.dot(h, wq2_ref[...], preferred_element_type=f32) + bq2_ref[...]
    qb = q
    kb = (jnp.dot(moves_ref[...], wkey_ref[...], preferred_element_type=f32)
          + bkey_ref[...])
    n_tiles = bt // tile
    iota_r = jax.lax.broadcasted_iota(jnp.int32, (tile, tile * n_moves), 0)
    iota_c = jax.lax.broadcasted_iota(jnp.int32, (tile, tile * n_moves), 1)
    band = (iota_c // n_moves) == iota_r
    tiles_out = []
    for t in range(n_tiles):
        qt = qb[t * tile:(t + 1) * tile, :]
        kt = kb[t * tile * n_moves:(t + 1) * tile * n_moves, :]
        d = jax.lax.dot_general(qt, kt, (((1,), (1,)), ((), ())),
                                preferred_element_type=f32)
        x = jnp.where(band, d, 0.0)
        shift = n_moves
        while shift < tile * n_moves:
            x = x + jnp.roll(x, -shift, axis=1)
            shift *= 2
        tiles_out.append(x[:, :n_moves])
    logits = jnp.concatenate(tiles_out, axis=0)
    logits_ref[...] = logits
    mask = mask_ref[...] > 0.0
    masked = jnp.where(mask, logits, -1e30)
    masked = masked - jnp.max(masked, axis=-1, keepdims=True)
    exps = jnp.where(mask, jnp.exp(masked), 0.0)
    policy = exps / jnp.sum(exps, axis=-1, keepdims=True)
    policy_ref[...] = policy
    score = jnp.log(policy + 1e-30) + g_ref[...]
    best = jnp.max(score, axis=-1, keepdims=True)
    ids = jax.lax.broadcasted_iota(jnp.int32, score.shape, 1)
    idx = jnp.min(jnp.where(score >= best, ids, n_moves), axis=-1,
                  keepdims=True)
    idx_ref[...] = idx


def kernel(action_type_index, autoregressive_embedding, max_moves,
           max_move_mask, W_key, b_key, W_q1, b_q1, W_q2, b_q2,
           W_proj, b_proj):
    T, B, S = autoregressive_embedding.shape
    N = max_moves.shape[-2]
    E = max_moves.shape[-1]
    K = W_key.shape[-1]
    TB = T * B

    are = autoregressive_embedding.reshape(TB, S)
    moves = max_moves.reshape(TB * N, E)
    wq1 = W_q1
    wq2 = W_q2
    wkey = W_key
    mask = max_move_mask.reshape(TB, N)
    mask = jnp.where(jnp.sum(mask) == 0, jnp.ones_like(mask), mask)
    mask_f = mask.astype(jnp.float32)
    gumbel = jax.random.gumbel(jax.random.key(42), (TB, N), jnp.float32)
    bq1 = b_q1.reshape(1, K)
    bq2 = b_q2.reshape(1, K)
    bkey = b_key.reshape(1, K)

    BT = 128
    TILE = 128
    grid = (TB // BT,)

    tok = lambda i: (i, 0)
    rep = lambda i: (0, 0)

    logits, policy, idx = pl.pallas_call(
        functools.partial(_mmh_block, n_moves=N, tile=TILE),
        grid=grid,
        in_specs=[
            pl.BlockSpec((BT, S), tok),
            pl.BlockSpec((BT * N, E), tok),
            pl.BlockSpec((BT, N), tok),
            pl.BlockSpec((BT, N), tok),
            pl.BlockSpec((S, K), rep),
            pl.BlockSpec((1, K), rep),
            pl.BlockSpec((K, K), rep),
            pl.BlockSpec((1, K), rep),
            pl.BlockSpec((E, K), rep),
            pl.BlockSpec((1, K), rep),
        ],
        out_specs=[
            pl.BlockSpec((BT, N), tok),
            pl.BlockSpec((BT, N), tok),
            pl.BlockSpec((BT, 1), tok),
        ],
        out_shape=[
            jax.ShapeDtypeStruct((TB, N), jnp.float32),
            jax.ShapeDtypeStruct((TB, N), jnp.float32),
            jax.ShapeDtypeStruct((TB, 1), jnp.int32),
        ],
        compiler_params=pltpu.CompilerParams(
            dimension_semantics=("parallel",),
        ),
    )(are, moves, mask_f, gumbel, wq1, bq1, wq2, bq2, wkey, bkey)

    return (logits.reshape(T, B, N), policy.reshape(T, B, N),
            idx.reshape(T, B, 1))


# speedup vs baseline: 1.0496x; 1.0496x over previous
import functools

import jax
import jax.numpy as jnp
from jax.experimental import pallas as pl
from jax.experimental.pallas import tpu as pltpu


def _mmh_block(are_ref, moves_ref, mask_ref, g_ref,
               wq1_ref, bq1_ref, wq2_ref, bq2_ref, wkey_ref, bkey_ref,
               logits_ref, policy_ref, idx_ref, *, n_moves, tile):
    f32 = jnp.float32
    bt = logits_ref.shape[0]
    h = jnp.dot(are_ref[...], wq1_ref[...], preferred_element_type=f32)
    h = jnp.maximum(h + bq1_ref[...], 0.0)
    q = jnp.dot(h, wq2_ref[...], preferred_element_type=f32) + bq2_ref[...]
    qb = q
    kb = (jnp.dot(moves_ref[...], wkey_ref[...], preferred_element_type=f32)
          + bkey_ref[...])
    n_tiles = bt // tile
    iota_r = jax.lax.broadcasted_iota(jnp.int32, (tile, tile * n_moves), 0)
    iota_c = jax.lax.broadcasted_iota(jnp.int32, (tile, tile * n_moves), 1)
    band = (iota_c // n_moves) == iota_r
    tiles_out = []
    for t in range(n_tiles):
        qt = qb[t * tile:(t + 1) * tile, :]
        kt = kb[t * tile * n_moves:(t + 1) * tile * n_moves, :]
        d = jax.lax.dot_general(qt, kt, (((1,), (1,)), ((), ())),
                                preferred_element_type=f32)
        x = jnp.where(band, d, 0.0)
        shift = n_moves
        while shift < tile * n_moves:
            x = x + jnp.roll(x, -shift, axis=1)
            shift *= 2
        tiles_out.append(x[:, :n_moves])
    logits = jnp.concatenate(tiles_out, axis=0)
    logits_ref[...] = logits
    mask = mask_ref[...] > 0.0
    masked = jnp.where(mask, logits, -1e30)
    masked = masked - jnp.max(masked, axis=-1, keepdims=True)
    exps = jnp.where(mask, jnp.exp(masked), 0.0)
    policy = exps / jnp.sum(exps, axis=-1, keepdims=True)
    policy_ref[...] = policy
    score = jnp.log(policy + 1e-30) + g_ref[...]
    best = jnp.max(score, axis=-1, keepdims=True)
    ids = jax.lax.broadcasted_iota(jnp.int32, score.shape, 1)
    idx = jnp.min(jnp.where(score >= best, ids, n_moves), axis=-1,
                  keepdims=True)
    idx_ref[...] = idx


def kernel(action_type_index, autoregressive_embedding, max_moves,
           max_move_mask, W_key, b_key, W_q1, b_q1, W_q2, b_q2,
           W_proj, b_proj):
    T, B, S = autoregressive_embedding.shape
    N = max_moves.shape[-2]
    E = max_moves.shape[-1]
    K = W_key.shape[-1]
    TB = T * B

    are = autoregressive_embedding.reshape(TB, S)
    moves = max_moves.reshape(TB * N, E)
    wq1 = W_q1
    wq2 = W_q2
    wkey = W_key
    mask = max_move_mask.reshape(TB, N)
    mask = jnp.where(jnp.sum(mask) == 0, jnp.ones_like(mask), mask)
    mask_f = mask.astype(jnp.float32)
    gumbel = jax.random.gumbel(jax.random.key(42), (TB, N), jnp.float32)
    bq1 = b_q1.reshape(1, K)
    bq2 = b_q2.reshape(1, K)
    bkey = b_key.reshape(1, K)

    BT = 256
    TILE = 128
    grid = (TB // BT,)

    tok = lambda i: (i, 0)
    rep = lambda i: (0, 0)

    logits, policy, idx = pl.pallas_call(
        functools.partial(_mmh_block, n_moves=N, tile=TILE),
        grid=grid,
        in_specs=[
            pl.BlockSpec((BT, S), tok),
            pl.BlockSpec((BT * N, E), tok),
            pl.BlockSpec((BT, N), tok),
            pl.BlockSpec((BT, N), tok),
            pl.BlockSpec((S, K), rep),
            pl.BlockSpec((1, K), rep),
            pl.BlockSpec((K, K), rep),
            pl.BlockSpec((1, K), rep),
            pl.BlockSpec((E, K), rep),
            pl.BlockSpec((1, K), rep),
        ],
        out_specs=[
            pl.BlockSpec((BT, N), tok),
            pl.BlockSpec((BT, N), tok),
            pl.BlockSpec((BT, 1), tok),
        ],
        out_shape=[
            jax.ShapeDtypeStruct((TB, N), jnp.float32),
            jax.ShapeDtypeStruct((TB, N), jnp.float32),
            jax.ShapeDtypeStruct((TB, 1), jnp.int32),
        ],
        compiler_params=pltpu.CompilerParams(
            dimension_semantics=("parallel",),
        ),
    )(are, moves, mask_f, gumbel, wq1, bq1, wq2, bq2, wkey, bkey)

    return (logits.reshape(T, B, N), policy.reshape(T, B, N),
            idx.reshape(T, B, 1))


# speedup vs baseline: 1.1255x; 1.0724x over previous
import functools

import jax
import jax.numpy as jnp
import numpy as np
from jax.experimental import pallas as pl
from jax.experimental.pallas import tpu as pltpu


@functools.lru_cache(maxsize=None)
def _gumbel_const(tb, n):
    with jax.ensure_compile_time_eval():
        return np.asarray(
            jax.random.gumbel(jax.random.key(42), (tb, n), jnp.float32))


def _mmh_block(are_ref, moves_ref, mask_ref, g_ref,
               wq1_ref, bq1_ref, wq2_ref, bq2_ref, wkey_ref, bkey_ref,
               logits_ref, policy_ref, idx_ref, *, n_moves, tile):
    f32 = jnp.float32
    bt = logits_ref.shape[0]
    h = jnp.dot(are_ref[...], wq1_ref[...], preferred_element_type=f32)
    h = jnp.maximum(h + bq1_ref[...], 0.0)
    q = jnp.dot(h, wq2_ref[...], preferred_element_type=f32) + bq2_ref[...]
    qb = q
    kb = (jnp.dot(moves_ref[...], wkey_ref[...], preferred_element_type=f32)
          + bkey_ref[...])
    n_tiles = bt // tile
    iota_r = jax.lax.broadcasted_iota(jnp.int32, (tile, tile * n_moves), 0)
    iota_c = jax.lax.broadcasted_iota(jnp.int32, (tile, tile * n_moves), 1)
    band = (iota_c // n_moves) == iota_r
    tiles_out = []
    for t in range(n_tiles):
        qt = qb[t * tile:(t + 1) * tile, :]
        kt = kb[t * tile * n_moves:(t + 1) * tile * n_moves, :]
        d = jax.lax.dot_general(qt, kt, (((1,), (1,)), ((), ())),
                                preferred_element_type=f32)
        x = jnp.where(band, d, 0.0)
        shift = n_moves
        while shift < tile * n_moves:
            x = x + jnp.roll(x, -shift, axis=1)
            shift *= 2
        tiles_out.append(x[:, :n_moves])
    logits = jnp.concatenate(tiles_out, axis=0)
    logits_ref[...] = logits
    mask = mask_ref[...] > 0.0
    masked = jnp.where(mask, logits, -1e30)
    masked = masked - jnp.max(masked, axis=-1, keepdims=True)
    exps = jnp.where(mask, jnp.exp(masked), 0.0)
    policy = exps / jnp.sum(exps, axis=-1, keepdims=True)
    policy_ref[...] = policy
    score = jnp.log(policy + 1e-30) + g_ref[...]
    best = jnp.max(score, axis=-1, keepdims=True)
    ids = jax.lax.broadcasted_iota(jnp.int32, score.shape, 1)
    idx = jnp.min(jnp.where(score >= best, ids, n_moves), axis=-1,
                  keepdims=True)
    idx_ref[...] = idx


def kernel(action_type_index, autoregressive_embedding, max_moves,
           max_move_mask, W_key, b_key, W_q1, b_q1, W_q2, b_q2,
           W_proj, b_proj):
    T, B, S = autoregressive_embedding.shape
    N = max_moves.shape[-2]
    E = max_moves.shape[-1]
    K = W_key.shape[-1]
    TB = T * B

    are = autoregressive_embedding.reshape(TB, S)
    moves = max_moves.reshape(TB * N, E)
    wq1 = W_q1
    wq2 = W_q2
    wkey = W_key
    mask = max_move_mask.reshape(TB, N)
    mask = jnp.where(jnp.sum(mask) == 0, jnp.ones_like(mask), mask)
    mask_f = mask.astype(jnp.float32)
    gumbel = jnp.asarray(_gumbel_const(TB, N))
    bq1 = b_q1.reshape(1, K)
    bq2 = b_q2.reshape(1, K)
    bkey = b_key.reshape(1, K)

    BT = 256
    TILE = 128
    grid = (TB // BT,)

    tok = lambda i: (i, 0)
    rep = lambda i: (0, 0)

    logits, policy, idx = pl.pallas_call(
        functools.partial(_mmh_block, n_moves=N, tile=TILE),
        grid=grid,
        in_specs=[
            pl.BlockSpec((BT, S), tok),
            pl.BlockSpec((BT * N, E), tok),
            pl.BlockSpec((BT, N), tok),
            pl.BlockSpec((BT, N), tok),
            pl.BlockSpec((S, K), rep),
            pl.BlockSpec((1, K), rep),
            pl.BlockSpec((K, K), rep),
            pl.BlockSpec((1, K), rep),
            pl.BlockSpec((E, K), rep),
            pl.BlockSpec((1, K), rep),
        ],
        out_specs=[
            pl.BlockSpec((BT, N), tok),
            pl.BlockSpec((BT, N), tok),
            pl.BlockSpec((BT, 1), tok),
        ],
        out_shape=[
            jax.ShapeDtypeStruct((TB, N), jnp.float32),
            jax.ShapeDtypeStruct((TB, N), jnp.float32),
            jax.ShapeDtypeStruct((TB, 1), jnp.int32),
        ],
        compiler_params=pltpu.CompilerParams(
            dimension_semantics=("parallel",),
            vmem_limit_bytes=100 * 1024 * 1024,
        ),
    )(are, moves, mask_f, gumbel, wq1, bq1, wq2, bq2, wkey, bkey)

    return (logits.reshape(T, B, N), policy.reshape(T, B, N),
            idx.reshape(T, B, 1))


# speedup vs baseline: 1.1466x; 1.0188x over previous
import functools

import jax
import jax.numpy as jnp
import numpy as np
from jax.experimental import pallas as pl
from jax.experimental.pallas import tpu as pltpu


@functools.lru_cache(maxsize=None)
def _gumbel_const(tb, n):
    with jax.ensure_compile_time_eval():
        return np.asarray(
            jax.random.gumbel(jax.random.key(42), (tb, n), jnp.float32))


def _gumbel(tb, n):
    try:
        return jnp.asarray(_gumbel_const(tb, n))
    except Exception:
        return jax.random.gumbel(jax.random.key(42), (tb, n), jnp.float32)


def _mmh_block(are_ref, moves_ref, mask_ref, maskfull_ref, g_ref,
               wq1_ref, bq1_ref, wq2_ref, bq2_ref, wkey_ref, bkey_ref,
               logits_ref, policy_ref, idx_ref, *, n_moves, tile):
    f32 = jnp.float32
    bt = logits_ref.shape[0]
    h = jnp.dot(are_ref[...], wq1_ref[...], preferred_element_type=f32)
    h = jnp.maximum(h + bq1_ref[...], 0.0)
    q = jnp.dot(h, wq2_ref[...], preferred_element_type=f32) + bq2_ref[...]
    qb = q
    kb = (jnp.dot(moves_ref[...], wkey_ref[...], preferred_element_type=f32)
          + bkey_ref[...])
    n_tiles = bt // tile
    iota_r = jax.lax.broadcasted_iota(jnp.int32, (tile, tile * n_moves), 0)
    iota_c = jax.lax.broadcasted_iota(jnp.int32, (tile, tile * n_moves), 1)
    band = (iota_c // n_moves) == iota_r
    tiles_out = []
    for t in range(n_tiles):
        qt = qb[t * tile:(t + 1) * tile, :]
        kt = kb[t * tile * n_moves:(t + 1) * tile * n_moves, :]
        d = jax.lax.dot_general(qt, kt, (((1,), (1,)), ((), ())),
                                preferred_element_type=f32)
        x = jnp.where(band, d, 0.0)
        shift = n_moves
        while shift < tile * n_moves:
            x = x + jnp.roll(x, -shift, axis=1)
            shift *= 2
        tiles_out.append(x[:, :n_moves])
    logits = jnp.concatenate(tiles_out, axis=0)
    logits_ref[...] = logits
    any_set = jnp.sum(maskfull_ref[...].astype(jnp.float32)) > 0.0
    mask = jnp.logical_or(mask_ref[...].astype(jnp.float32) > 0.0,
                          jnp.logical_not(any_set))
    masked = jnp.where(mask, logits, -1e30)
    masked = masked - jnp.max(masked, axis=-1, keepdims=True)
    exps = jnp.where(mask, jnp.exp(masked), 0.0)
    policy = exps / jnp.sum(exps, axis=-1, keepdims=True)
    policy_ref[...] = policy
    score = jnp.log(policy + 1e-30) + g_ref[...]
    best = jnp.max(score, axis=-1, keepdims=True)
    ids = jax.lax.broadcasted_iota(jnp.int32, score.shape, 1)
    idx = jnp.min(jnp.where(score >= best, ids, n_moves), axis=-1,
                  keepdims=True)
    idx_ref[...] = idx


def kernel(action_type_index, autoregressive_embedding, max_moves,
           max_move_mask, W_key, b_key, W_q1, b_q1, W_q2, b_q2,
           W_proj, b_proj):
    T, B, S = autoregressive_embedding.shape
    N = max_moves.shape[-2]
    E = max_moves.shape[-1]
    K = W_key.shape[-1]
    TB = T * B

    are = autoregressive_embedding.reshape(TB, S)
    moves = max_moves.reshape(TB * N, E)
    wq1 = W_q1
    wq2 = W_q2
    wkey = W_key
    mask_i8 = max_move_mask.reshape(TB, N).astype(jnp.int8)
    gumbel = _gumbel(TB, N)
    bq1 = b_q1.reshape(1, K)
    bq2 = b_q2.reshape(1, K)
    bkey = b_key.reshape(1, K)

    BT = 256
    TILE = 128
    grid = (TB // BT,)

    tok = lambda i: (i, 0)
    rep = lambda i: (0, 0)

    logits, policy, idx = pl.pallas_call(
        functools.partial(_mmh_block, n_moves=N, tile=TILE),
        grid=grid,
        in_specs=[
            pl.BlockSpec((BT, S), tok),
            pl.BlockSpec((BT * N, E), tok),
            pl.BlockSpec((BT, N), tok),
            pl.BlockSpec((TB, N), rep),
            pl.BlockSpec((BT, N), tok),
            pl.BlockSpec((S, K), rep),
            pl.BlockSpec((1, K), rep),
            pl.BlockSpec((K, K), rep),
            pl.BlockSpec((1, K), rep),
            pl.BlockSpec((E, K), rep),
            pl.BlockSpec((1, K), rep),
        ],
        out_specs=[
            pl.BlockSpec((BT, N), tok),
            pl.BlockSpec((BT, N), tok),
            pl.BlockSpec((BT, 1), tok),
        ],
        out_shape=[
            jax.ShapeDtypeStruct((TB, N), jnp.float32),
            jax.ShapeDtypeStruct((TB, N), jnp.float32),
            jax.ShapeDtypeStruct((TB, 1), jnp.int32),
        ],
        compiler_params=pltpu.CompilerParams(
            dimension_semantics=("parallel",),
            vmem_limit_bytes=100 * 1024 * 1024,
        ),
    )(are, moves, mask_i8, mask_i8, gumbel, wq1, bq1, wq2, bq2, wkey, bkey)

    return (logits.reshape(T, B, N), policy.reshape(T, B, N),
            idx.reshape(T, B, 1))
